# Initial kernel scaffold; baseline (speedup 1.0000x reference)
import jax
import jax.numpy as jnp
from jax.experimental import pallas as pl


def kernel(x, edge_index, batch, W1, b1, g1, bt1, W2, b2, g2, bt2, W3, b3, g3, bt3, W4, b4, g4, bt4, W5, b5, g5, bt5, Wl, bl):
    raise NotImplementedError("write your pallas kernel here")


# speedup vs baseline: 12.9801x; 12.9801x over previous
import functools

import jax
import jax.numpy as jnp
from jax import lax
from jax.experimental import pallas as pl
from jax.experimental.pallas import tpu as pltpu
from jax.experimental.pallas import tpu_sc as plsc

N = 10000
E = 320000
F_IN = 128
H = 256
HH = 128
C = 10

NT = 16
E_PAD = 327680
K = 256
EPT = E_PAD // NT
NCH = EPT // K
HS = 32
NPASS = HH // HS
N_ACC = N + 8
RPT = 624
RLAST = N - 15 * RPT
NFILL = RLAST // 16
NB = 320

_mesh = plsc.VectorSubcoreMesh(core_axis_name="c", subcore_axis_name="s")
_BLK = 1000
_NBLK = N // _BLK


def _edge_body(srcr_hbm, dstr_hbm, zt_hbm, out_hbm,
               src_v, dst_v, rows0, rows1, rows2, rows3, nbuf, nidx,
               table_sh, acc_sh, gsem0, gsem1, gsem2, gsem3,
               ssem0, ssem1, ssem2, ssem3):
    rows = (rows0, rows1, rows2, rows3)
    gsem = (gsem0, gsem1, gsem2, gsem3)
    ssem = (ssem0, ssem1, ssem2, ssem3)
    sem0 = gsem0
    c = lax.axis_index("c")
    t = lax.axis_index("s")

    pltpu.sync_copy(srcr_hbm.at[t], src_v)
    pltpu.sync_copy(dstr_hbm.at[t], dst_v)
    base = jnp.minimum(t, 15) * RPT

    def pass_body(p, _):
        def fillb(i, _):
            lane = lax.iota(jnp.int32, 16)
            nidx[pl.ds(i * 16, 16)] = (
                NPASS * (c * N + base + i * 16 + lane) + p)
            return 0
        lax.fori_loop(0, NFILL, fillb, 0)

        def init_io(sizes):
            off = 0
            for sz in sizes:
                pltpu.async_copy(zt_hbm.at[nidx.at[pl.ds(off, sz)]],
                                 nbuf.at[pl.ds(0, sz)], sem0)
                pltpu.make_async_copy(zt_hbm.at[nidx.at[pl.ds(off, sz)]],
                                      nbuf.at[pl.ds(0, sz)], sem0).wait()
                pltpu.sync_copy(nbuf.at[pl.ds(0, sz)],
                                acc_sh.at[pl.ds(base + off, sz)])
                pltpu.sync_copy(nbuf.at[pl.ds(0, sz)],
                                table_sh.at[pl.ds(base + off, sz)])
                off += sz

        @pl.when(t < 15)
        def _():
            init_io((NB, RPT - NB))

        @pl.when(t == 15)
        def _():
            init_io((NB, RLAST - NB))

        plsc.subcore_barrier()

        def gsrc(j):
            return table_sh.at[src_v.at[pl.ds(j * K, K)]]

        def sdst(j):
            return acc_sh.at[dst_v.at[pl.ds(j * K, K)]]

        pltpu.async_copy(gsrc(0), rows[0], gsem[0])
        pltpu.async_copy(gsrc(1), rows[1], gsem[1])

        def body(g, _):
            for b in range(4):
                j = 4 * g + b
                b2 = (b + 2) % 4
                pltpu.make_async_copy(gsrc(j), rows[b], gsem[b]).wait()
                pltpu.async_copy(rows[b], sdst(j), ssem[b], add=True)

                @pl.when(j >= 2)
                def _():
                    pltpu.make_async_copy(rows[b2], sdst(j - 2),
                                          ssem[b2]).wait()

                @pl.when(j + 2 < NCH)
                def _():
                    pltpu.async_copy(gsrc(j + 2), rows[b2], gsem[b2])
            return 0

        lax.fori_loop(0, NCH // 4, body, 0)
        pltpu.make_async_copy(rows[2], sdst(NCH - 2), ssem[2]).wait()
        pltpu.make_async_copy(rows[3], sdst(NCH - 1), ssem[3]).wait()

        plsc.subcore_barrier()

        def write_io(sizes):
            off = 0
            for sz in sizes:
                pltpu.sync_copy(acc_sh.at[pl.ds(base + off, sz)],
                                nbuf.at[pl.ds(0, sz)])
                pltpu.async_copy(nbuf.at[pl.ds(0, sz)],
                                 out_hbm.at[nidx.at[pl.ds(off, sz)]], sem0)
                pltpu.make_async_copy(nbuf.at[pl.ds(0, sz)],
                                      out_hbm.at[nidx.at[pl.ds(off, sz)]],
                                      sem0).wait()
                off += sz

        @pl.when(t < 15)
        def _():
            write_io((NB, RPT - NB))

        @pl.when(t == 15)
        def _():
            write_io((NB, RLAST - NB))

        return 0

    lax.fori_loop(0, NPASS, pass_body, 0)


_edge_call = functools.partial(
    pl.kernel,
    mesh=_mesh,
    out_type=jax.ShapeDtypeStruct((2 * NPASS * N, HS), jnp.float32),
    scratch_types=[
        pltpu.VMEM((EPT,), jnp.int32),
        pltpu.VMEM((EPT,), jnp.int32),
        pltpu.VMEM((K, HS), jnp.float32),
        pltpu.VMEM((K, HS), jnp.float32),
        pltpu.VMEM((K, HS), jnp.float32),
        pltpu.VMEM((K, HS), jnp.float32),
        pltpu.VMEM((NB, HS), jnp.float32),
        pltpu.VMEM((RLAST,), jnp.int32),
        pltpu.VMEM_SHARED((N_ACC, HS), jnp.float32),
        pltpu.VMEM_SHARED((N_ACC, HS), jnp.float32),
        pltpu.SemaphoreType.DMA,
        pltpu.SemaphoreType.DMA,
        pltpu.SemaphoreType.DMA,
        pltpu.SemaphoreType.DMA,
        pltpu.SemaphoreType.DMA,
        pltpu.SemaphoreType.DMA,
        pltpu.SemaphoreType.DMA,
        pltpu.SemaphoreType.DMA,
    ],
    compiler_params=pltpu.CompilerParams(use_tc_tiling_on_sc=False),
)(_edge_body)


_DOT = dict(preferred_element_type=jnp.float32, precision=lax.Precision.HIGHEST)


def _prep_body(dego_ref, x_ref, w_ref, b_ref, dis_ref, zt_ref):
    deg = dego_ref[0, :, 0, 0:1]
    dis = lax.rsqrt(deg)
    dis_ref[...] = dis
    z = jnp.dot(x_ref[...], w_ref[...], **_DOT) + b_ref[...]
    zt = z * dis
    zt_ref[0] = zt[:, :HH]
    zt_ref[1] = zt[:, HH:]


def _tc_prep(dego, x, w1, b1):
    return pl.pallas_call(
        _prep_body,
        grid=(_NBLK,),
        in_specs=[
            pl.BlockSpec((1, _BLK, NPASS, HS), lambda i: (0, i, 0, 0)),
            pl.BlockSpec((_BLK, F_IN), lambda i: (i, 0)),
            pl.BlockSpec((F_IN, H), lambda i: (0, 0)),
            pl.BlockSpec((1, H), lambda i: (0, 0)),
        ],
        out_specs=[
            pl.BlockSpec((_BLK, 1), lambda i: (i, 0)),
            pl.BlockSpec((2, _BLK, HH), lambda i: (0, i, 0)),
        ],
        out_shape=[
            jax.ShapeDtypeStruct((N, 1), jnp.float32),
            jax.ShapeDtypeStruct((2, N, HH), jnp.float32),
        ],
    )(dego, x, w1, b1)


def _mid_body(acc_ref, dis_ref, y_ref, s_ref, q_ref):
    i = pl.program_id(0)
    y = jnp.concatenate([acc_ref[0], acc_ref[1]], axis=1) * dis_ref[...]
    y_ref[...] = y
    ps = jnp.sum(y, 0, keepdims=True)
    pq = jnp.sum(y * y, 0, keepdims=True)

    @pl.when(i == 0)
    def _():
        s_ref[...] = ps
        q_ref[...] = pq

    @pl.when(i > 0)
    def _():
        s_ref[...] += ps
        q_ref[...] += pq


def _tc_mid(acc, dis):
    return pl.pallas_call(
        _mid_body,
        grid=(_NBLK,),
        in_specs=[
            pl.BlockSpec((2, _BLK, HH), lambda i: (0, i, 0)),
            pl.BlockSpec((_BLK, 1), lambda i: (i, 0)),
        ],
        out_specs=[
            pl.BlockSpec((_BLK, H), lambda i: (i, 0)),
            pl.BlockSpec((1, H), lambda i: (0, 0)),
            pl.BlockSpec((1, H), lambda i: (0, 0)),
        ],
        out_shape=[
            jax.ShapeDtypeStruct((N, H), jnp.float32),
            jax.ShapeDtypeStruct((1, H), jnp.float32),
            jax.ShapeDtypeStruct((1, H), jnp.float32),
        ],
    )(acc, dis)


def _fused_body(y_ref, s_ref, q_ref, g_ref, bt_ref, w_ref, b_ref, dis_ref, zt_ref):
    m = s_ref[...] / N
    v = q_ref[...] / N - m * m
    inv = lax.rsqrt(v + 1e-5)
    h = jnp.maximum((y_ref[...] - m) * inv * g_ref[...] + bt_ref[...], 0.0)
    z = jnp.dot(h, w_ref[...], **_DOT) + b_ref[...]
    zt = z * dis_ref[...]
    zt_ref[0] = zt[:, :HH]
    zt_ref[1] = zt[:, HH:]


def _tc_fused(y, s, q, g, bt, w, b, dis):
    return pl.pallas_call(
        _fused_body,
        grid=(_NBLK,),
        in_specs=[
            pl.BlockSpec((_BLK, H), lambda i: (i, 0)),
            pl.BlockSpec((1, H), lambda i: (0, 0)),
            pl.BlockSpec((1, H), lambda i: (0, 0)),
            pl.BlockSpec((1, H), lambda i: (0, 0)),
            pl.BlockSpec((1, H), lambda i: (0, 0)),
            pl.BlockSpec((H, H), lambda i: (0, 0)),
            pl.BlockSpec((1, H), lambda i: (0, 0)),
            pl.BlockSpec((_BLK, 1), lambda i: (i, 0)),
        ],
        out_specs=pl.BlockSpec((2, _BLK, HH), lambda i: (0, i, 0)),
        out_shape=jax.ShapeDtypeStruct((2, N, HH), jnp.float32),
    )(y, s, q, g, bt, w, b, dis)


def _final_body(y_ref, s_ref, q_ref, g_ref, bt_ref, wl_ref, bl_ref, o_ref, acc_ref):
    i = pl.program_id(0)
    m = s_ref[...] / N
    v = q_ref[...] / N - m * m
    inv = lax.rsqrt(v + 1e-5)
    h = jnp.maximum((y_ref[...] - m) * inv * g_ref[...] + bt_ref[...], 0.0)
    ps = jnp.sum(h, 0, keepdims=True)

    @pl.when(i == 0)
    def _():
        acc_ref[...] = ps

    @pl.when(i > 0)
    def _():
        acc_ref[...] += ps

    @pl.when(i == _NBLK - 1)
    def _():
        pooled = acc_ref[...] / N
        o_ref[...] = jnp.dot(pooled, wl_ref[...], **_DOT) + bl_ref[...]


def _tc_final(y, s, q, g, bt, wlp, blp):
    return pl.pallas_call(
        _final_body,
        grid=(_NBLK,),
        in_specs=[
            pl.BlockSpec((_BLK, H), lambda i: (i, 0)),
            pl.BlockSpec((1, H), lambda i: (0, 0)),
            pl.BlockSpec((1, H), lambda i: (0, 0)),
            pl.BlockSpec((1, H), lambda i: (0, 0)),
            pl.BlockSpec((1, H), lambda i: (0, 0)),
            pl.BlockSpec((H, HH), lambda i: (0, 0)),
            pl.BlockSpec((1, HH), lambda i: (0, 0)),
        ],
        out_specs=pl.BlockSpec((1, HH), lambda i: (0, 0)),
        out_shape=jax.ShapeDtypeStruct((1, HH), jnp.float32),
        scratch_shapes=[pltpu.VMEM((1, H), jnp.float32)],
    )(y, s, q, g, bt, wlp, blp)


def kernel(x, edge_index, batch, W1, b1, g1, bt1, W2, b2, g2, bt2, W3, b3, g3,
           bt3, W4, b4, g4, bt4, W5, b5, g5, bt5, Wl, bl):
    src = edge_index[0]
    dst = edge_index[1]

    pad = E_PAD - E
    src_p = jnp.concatenate([src, jnp.zeros((pad,), src.dtype)])
    dst_p = jnp.concatenate([dst, jnp.full((pad,), N, dst.dtype)])
    srcr = src_p.reshape(NT, EPT)
    dstr = dst_p.reshape(NT, EPT)

    ones_t = jnp.ones((2 * NPASS * N, HS), jnp.float32)
    dego = _edge_call(srcr, dstr, ones_t)
    dis, zt = _tc_prep(dego.reshape(2, N, NPASS, HS), x, W1,
                       b1.reshape(1, H))

    params = [(W2, b2, g2, bt2), (W3, b3, g3, bt3), (W4, b4, g4, bt4),
              (W5, b5, g5, bt5)]
    y = None
    s = q = None
    for li in range(5):
        acc = _edge_call(srcr, dstr, zt.reshape(2 * NPASS * N, HS))
        y, s, q = _tc_mid(acc.reshape(2, N, HH), dis)
        if li < 4:
            W, b, g, bt = params[li]
            zt = _tc_fused(y, s, q, g.reshape(1, H), bt.reshape(1, H), W,
                           b.reshape(1, H), dis)

    wlp = jnp.pad(Wl, ((0, 0), (0, HH - C)))
    blp = jnp.pad(bl, (0, HH - C)).reshape(1, HH)
    o = _tc_final(y, s, q, g5.reshape(1, H), bt5.reshape(1, H), wlp, blp)
    return o[:, :C]
